# baseline (speedup 1.0000x reference)
import functools

import jax
import jax.numpy as jnp
import numpy as np
from jax import lax
from jax.experimental import pallas as pl
from jax.experimental.pallas import tpu as pltpu
from jax.experimental.pallas import tpu_sc as plsc

N = 10000
D = 128
H = 64
E = 320000
B = 128
GT = 160
GW = 80
EP = GT * 16 * B
GB = N
ACCN = 10240
SST = 20
SFLAT = ACCN * SST
NBUF = 2
CH = 128
NCH = EP // 32 // CH
NBLK = 10
RB = N // NBLK

_mesh = plsc.VectorSubcoreMesh(
    core_axis_name="c", subcore_axis_name="s", num_cores=2, num_subcores=16
)
_sc_params = pltpu.CompilerParams(
    needs_layout_passes=False, use_tc_tiling_on_sc=False
)


@functools.partial(
    pl.kernel,
    out_type=jax.ShapeDtypeStruct((2 * ACCN,), jnp.float32),
    mesh=_mesh,
    compiler_params=_sc_params,
    scratch_types=[
        pltpu.VMEM((GW, B), jnp.int32),
        pltpu.VMEM((B,), jnp.float32),
        pltpu.VMEM((640,), jnp.float32),
        pltpu.VMEM_SHARED((ACCN,), jnp.float32),
    ],
)
def _sc_deg(srcd_hbm, degp_hbm, sbuf, ones_v, zv, acc):
    c = lax.axis_index("c")
    s = lax.axis_index("s")
    w = c * 16 + s

    def fill_ones(i, _):
        ones_v[pl.ds(i * 16, 16)] = jnp.full((16,), 1.0, jnp.float32)
        return 0

    def fill_zero(i, _):
        zv[pl.ds(i * 16, 16)] = jnp.zeros((16,), jnp.float32)
        return 0

    lax.fori_loop(0, 8, fill_ones, 0)
    lax.fori_loop(0, 40, fill_zero, 0)
    pltpu.sync_copy(zv, acc.at[pl.ds(s * 640, 640)])
    pltpu.sync_copy(srcd_hbm.at[pl.ds(w * GW, GW)], sbuf)
    plsc.subcore_barrier()

    def it(g, _):
        pltpu.sync_copy(ones_v, acc.at[sbuf.at[g]], add=True)
        return 0

    lax.fori_loop(0, GW, it, 0)
    plsc.subcore_barrier()
    pltpu.sync_copy(acc.at[pl.ds(s * 640, 640)],
                    degp_hbm.at[pl.ds(c * ACCN + s * 640, 640)])


@functools.partial(
    pl.kernel,
    out_type=jax.ShapeDtypeStruct((2 * SFLAT,), jnp.float32),
    mesh=_mesh,
    compiler_params=_sc_params,
    scratch_types=[
        pltpu.VMEM((GW, B), jnp.int32),
        pltpu.VMEM((GW, B), jnp.int32),
        pltpu.VMEM((GW, B), jnp.int32),
        pltpu.VMEM((GW, B), jnp.int32),
        pltpu.VMEM((ACCN,), jnp.float32),
        pltpu.VMEM((B,), jnp.float32),
        pltpu.VMEM((3200,), jnp.float32),
        pltpu.VMEM_SHARED((SFLAT,), jnp.float32),
    ],
)
def _sc_sprime(dis_hbm, srcg_hbm, dst_hbm, a0_hbm, a1_hbm, spart_hbm,
               sb, db, a0b, a1b, dis_t, pay, zf, accs):
    c = lax.axis_index("c")
    s = lax.axis_index("s")
    w = c * 16 + s

    def zfill(i, _):
        zf[pl.ds(i * 16, 16)] = jnp.zeros((16,), jnp.float32)
        return 0

    lax.fori_loop(0, 200, zfill, 0)
    for j in range(4):
        pltpu.sync_copy(zf, accs.at[pl.ds(s * 12800 + j * 3200, 3200)])
    pltpu.sync_copy(srcg_hbm.at[pl.ds(w * GW, GW)], sb)
    pltpu.sync_copy(dst_hbm.at[pl.ds(w * GW, GW)], db)
    pltpu.sync_copy(a0_hbm.at[pl.ds(w * GW, GW)], a0b)
    pltpu.sync_copy(a1_hbm.at[pl.ds(w * GW, GW)], a1b)
    pltpu.sync_copy(dis_hbm, dis_t)
    plsc.subcore_barrier()

    def it(g, _):
        for k in range(8):
            sl = pl.ds(k * 16, 16)
            s16 = sb[g, sl]
            d16 = db[g, sl]
            c16 = a0b[g, sl] * 3 + a1b[g, sl]
            pay[sl] = plsc.load_gather(dis_t, [s16])
            a0b[g, sl] = d16 * SST + c16
        pltpu.sync_copy(pay, accs.at[a0b.at[g]], add=True)
        return 0

    lax.fori_loop(0, GW, it, 0)
    plsc.subcore_barrier()
    pltpu.sync_copy(accs.at[pl.ds(s * 12800, 12800)],
                    spart_hbm.at[pl.ds(c * SFLAT + s * 12800, 12800)])


@functools.partial(
    pl.kernel,
    out_type=jax.ShapeDtypeStruct((2, ACCN, D), jnp.float32),
    mesh=_mesh,
    compiler_params=_sc_params,
    scratch_types=[
        pltpu.VMEM((NBUF, 2, CH), jnp.int32),
        pltpu.VMEM((NBUF, CH, D), jnp.float32),
        pltpu.VMEM_SHARED((ACCN, D), jnp.float32),
        pltpu.SemaphoreType.DMA((NBUF,)),
        pltpu.SemaphoreType.DMA((NBUF,)),
        pltpu.SemaphoreType.DMA((NBUF,)),
    ],
)
def _sc_rowprop(u_hbm, idx_hbm, vout_hbm, ibuf, rows, accv, gsem, ssem, isem):
    c = lax.axis_index("c")
    s = lax.axis_index("s")
    w = c * 16 + s

    def zrow(i, _):
        for k in range(D // 16):
            rows[0, i, pl.ds(k * 16, 16)] = jnp.zeros((16,), jnp.float32)
        return 0

    lax.fori_loop(0, CH, zrow, 0)
    for j in range(5):
        pltpu.sync_copy(rows.at[0], accv.at[pl.ds(s * 640 + j * 128, 128)])
    plsc.subcore_barrier()

    for b in range(NBUF):
        pltpu.sync_copy(idx_hbm.at[w * NCH + b], ibuf.at[b])
        pltpu.make_async_copy(u_hbm.at[ibuf.at[b, 0]], rows.at[b],
                              gsem.at[b]).start()

    def round_body(gi, _):
        for b in range(NBUF):
            pltpu.make_async_copy(u_hbm.at[ibuf.at[b, 0]], rows.at[b],
                                  gsem.at[b]).wait()
            pltpu.make_async_copy(rows.at[b], accv.at[ibuf.at[b, 1]],
                                  ssem.at[b]).start(add=True)
        for b in range(NBUF):
            gn = w * NCH + gi * NBUF + b + NBUF
            pltpu.make_async_copy(rows.at[b], accv.at[ibuf.at[b, 1]],
                                  ssem.at[b]).wait()
            pltpu.make_async_copy(idx_hbm.at[gn], ibuf.at[b],
                                  isem.at[b]).start()
        for b in range(NBUF):
            gn = w * NCH + (gi + 1) * NBUF + b
            pltpu.make_async_copy(idx_hbm.at[gn], ibuf.at[b],
                                  isem.at[b]).wait()
            pltpu.make_async_copy(u_hbm.at[ibuf.at[b, 0]], rows.at[b],
                                  gsem.at[b]).start()
        return 0

    lax.fori_loop(0, NCH // NBUF - 1, round_body, 0)
    for b in range(NBUF):
        pltpu.make_async_copy(u_hbm.at[ibuf.at[b, 0]], rows.at[b],
                              gsem.at[b]).wait()
        pltpu.make_async_copy(rows.at[b], accv.at[ibuf.at[b, 1]],
                              ssem.at[b]).start(add=True)
    for b in range(NBUF):
        pltpu.make_async_copy(rows.at[b], accv.at[ibuf.at[b, 1]],
                              ssem.at[b]).wait()
    plsc.subcore_barrier()
    pltpu.sync_copy(accv.at[pl.ds(s * 640, 640)],
                    vout_hbm.at[c, pl.ds(s * 640, 640)])


def _tc_dis_body(degp_ref, dis_ref):
    deg = degp_ref[0] + degp_ref[1]
    dis_ref[...] = jnp.where(deg > 0, lax.rsqrt(deg), 0.0)


_tc_dis = pl.pallas_call(
    _tc_dis_body,
    out_shape=jax.ShapeDtypeStruct((80, 128), jnp.float32),
    in_specs=[pl.BlockSpec((2, 80, 128), lambda: (0, 0, 0))],
    out_specs=pl.BlockSpec((80, 128), lambda: (0, 0)),
)


def _tc_u1_body(dis_ref, x_ref, u1_ref):
    u1_ref[...] = dis_ref[...] * x_ref[...]


_tc_u1 = pl.pallas_call(
    _tc_u1_body,
    grid=(NBLK,),
    out_shape=jax.ShapeDtypeStruct((N, D), jnp.float32),
    in_specs=[
        pl.BlockSpec((RB, 1), lambda b: (b, 0)),
        pl.BlockSpec((RB, D), lambda b: (b, 0)),
    ],
    out_specs=pl.BlockSpec((RB, D), lambda b: (b, 0)),
)


def _tc_mid_body(v10_ref, v11_ref, s0_ref, s1_ref, dis_ref, t_ref,
                 tx1_ref, u2_ref, b_ref):
    dis = dis_ref[...]
    bb = jnp.dot(s0_ref[...] + s1_ref[...], t_ref[...],
                 preferred_element_type=jnp.float32)
    b = dis * bb
    tx1 = -(dis * (v10_ref[...] + v11_ref[...]) + b)
    tx1_ref[...] = tx1
    u2_ref[...] = dis * tx1
    b_ref[...] = b


_tc_mid = pl.pallas_call(
    _tc_mid_body,
    grid=(NBLK,),
    out_shape=(
        jax.ShapeDtypeStruct((N, D), jnp.float32),
        jax.ShapeDtypeStruct((N, D), jnp.float32),
        jax.ShapeDtypeStruct((N, D), jnp.float32),
    ),
    in_specs=[
        pl.BlockSpec((RB, D), lambda b: (b, 0)),
        pl.BlockSpec((RB, D), lambda b: (b, 0)),
        pl.BlockSpec((RB, SST), lambda b: (b, 0)),
        pl.BlockSpec((RB, SST), lambda b: (b, 0)),
        pl.BlockSpec((RB, 1), lambda b: (b, 0)),
        pl.BlockSpec((SST, D), lambda b: (0, 0)),
    ],
    out_specs=(
        pl.BlockSpec((RB, D), lambda b: (b, 0)),
        pl.BlockSpec((RB, D), lambda b: (b, 0)),
        pl.BlockSpec((RB, D), lambda b: (b, 0)),
    ),
)


def _tc_final_body(x_ref, tx1_ref, v20_ref, v21_ref, b_ref, dis_ref, w_ref,
                   bias_ref, out_ref):
    x = x_ref[...]
    tx1 = tx1_ref[...]
    tx2 = -2.0 * (dis_ref[...] * (v20_ref[...] + v21_ref[...])
                  + b_ref[...]) - x
    acc = jnp.dot(x, w_ref[0], preferred_element_type=jnp.float32)
    acc += jnp.dot(tx1, w_ref[1], preferred_element_type=jnp.float32)
    acc += jnp.dot(tx2, w_ref[2], preferred_element_type=jnp.float32)
    out_ref[...] = acc + bias_ref[...]


_tc_final = pl.pallas_call(
    _tc_final_body,
    grid=(NBLK,),
    out_shape=jax.ShapeDtypeStruct((N, D), jnp.float32),
    in_specs=[
        pl.BlockSpec((RB, D), lambda b: (b, 0)),
        pl.BlockSpec((RB, D), lambda b: (b, 0)),
        pl.BlockSpec((RB, D), lambda b: (b, 0)),
        pl.BlockSpec((RB, D), lambda b: (b, 0)),
        pl.BlockSpec((RB, D), lambda b: (b, 0)),
        pl.BlockSpec((RB, 1), lambda b: (b, 0)),
        pl.BlockSpec((3, D, D), lambda b: (0, 0, 0)),
        pl.BlockSpec((1, D), lambda b: (0, 0)),
    ],
    out_specs=pl.BlockSpec((RB, D), lambda b: (b, 0)),
)


def kernel(x, edge_index, edge_attr, weight, bias, emb1, emb2):
    src = edge_index[0]
    dst = edge_index[1]
    a0 = edge_attr[:, 0]
    a1 = edge_attr[:, 1]
    P = EP - E
    zpad = jnp.zeros((P,), jnp.int32)
    gpad = jnp.full((P,), GB, jnp.int32)
    srcg = jnp.concatenate([src, zpad]).reshape(EP // B, B)
    srcd = jnp.concatenate([src, gpad]).reshape(EP // B, B)
    dstp = jnp.concatenate([dst, gpad]).reshape(EP // B, B)
    a0p = jnp.concatenate([a0, zpad]).reshape(EP // B, B)
    a1p = jnp.concatenate([a1, zpad]).reshape(EP // B, B)
    t_full = (emb1[:, None, :] + emb2[None, :, :]).reshape(18, D)
    tpad = jnp.concatenate([t_full, jnp.zeros((SST - 18, D), jnp.float32)],
                           axis=0)

    degp = _sc_deg(srcd)
    disf = _tc_dis(degp.reshape(2, 80, 128))
    dis_flat = disf.reshape(ACCN)
    discol = dis_flat[:N][:, None]
    u1 = _tc_u1(discol, x)
    spart = _sc_sprime(dis_flat, srcg, dstp, a0p, a1p)
    s2 = spart.reshape(2, ACCN, SST)
    packed = jnp.stack([srcg.reshape(-1, CH), dstp.reshape(-1, CH)], axis=1)
    v1p = _sc_rowprop(u1, packed)
    tx1, u2, b = _tc_mid(v1p[0, :N], v1p[1, :N], s2[0, :N], s2[1, :N],
                         discol, tpad)
    v2p = _sc_rowprop(u2, packed)
    out = _tc_final(x, tx1, v2p[0, :N], v2p[1, :N], b, discol, weight,
                    bias[None, :])
    return out


# speedup vs baseline: 1.3557x; 1.3557x over previous
import functools

import jax
import jax.numpy as jnp
import numpy as np
from jax import lax
from jax.experimental import pallas as pl
from jax.experimental.pallas import tpu as pltpu
from jax.experimental.pallas import tpu_sc as plsc

N = 10000
D = 128
H = 64
E = 320000
B = 128
GT = 160
GW = 80
EP = GT * 16 * B
GB = N
ACCN = 10240
SST = 20
SFLAT = ACCN * SST
NBUF = 4
CH = 256
NCH = EP // 16 // CH
NBLK = 10
RB = N // NBLK

_mesh = plsc.VectorSubcoreMesh(
    core_axis_name="c", subcore_axis_name="s", num_cores=2, num_subcores=16
)
_sc_params = pltpu.CompilerParams(
    needs_layout_passes=False, use_tc_tiling_on_sc=False
)


@functools.partial(
    pl.kernel,
    out_type=jax.ShapeDtypeStruct((2 * ACCN,), jnp.float32),
    mesh=_mesh,
    compiler_params=_sc_params,
    scratch_types=[
        pltpu.VMEM((GW, B), jnp.int32),
        pltpu.VMEM((B,), jnp.float32),
        pltpu.VMEM((640,), jnp.float32),
        pltpu.VMEM_SHARED((ACCN,), jnp.float32),
    ],
)
def _sc_deg(srcd_hbm, degp_hbm, sbuf, ones_v, zv, acc):
    c = lax.axis_index("c")
    s = lax.axis_index("s")
    w = c * 16 + s

    def fill_ones(i, _):
        ones_v[pl.ds(i * 16, 16)] = jnp.full((16,), 1.0, jnp.float32)
        return 0

    def fill_zero(i, _):
        zv[pl.ds(i * 16, 16)] = jnp.zeros((16,), jnp.float32)
        return 0

    lax.fori_loop(0, 8, fill_ones, 0)
    lax.fori_loop(0, 40, fill_zero, 0)
    pltpu.sync_copy(zv, acc.at[pl.ds(s * 640, 640)])
    pltpu.sync_copy(srcd_hbm.at[pl.ds(w * GW, GW)], sbuf)
    plsc.subcore_barrier()

    def it(g, _):
        pltpu.sync_copy(ones_v, acc.at[sbuf.at[g]], add=True)
        return 0

    lax.fori_loop(0, GW, it, 0)
    plsc.subcore_barrier()
    pltpu.sync_copy(acc.at[pl.ds(s * 640, 640)],
                    degp_hbm.at[pl.ds(c * ACCN + s * 640, 640)])


@functools.partial(
    pl.kernel,
    out_type=jax.ShapeDtypeStruct((2 * SFLAT,), jnp.float32),
    mesh=_mesh,
    compiler_params=_sc_params,
    scratch_types=[
        pltpu.VMEM((GW, B), jnp.int32),
        pltpu.VMEM((GW, B), jnp.int32),
        pltpu.VMEM((GW, B), jnp.int32),
        pltpu.VMEM((GW, B), jnp.int32),
        pltpu.VMEM((ACCN,), jnp.float32),
        pltpu.VMEM((B,), jnp.float32),
        pltpu.VMEM((3200,), jnp.float32),
        pltpu.VMEM_SHARED((SFLAT,), jnp.float32),
    ],
)
def _sc_sprime(dis_hbm, srcg_hbm, dst_hbm, a0_hbm, a1_hbm, spart_hbm,
               sb, db, a0b, a1b, dis_t, pay, zf, accs):
    c = lax.axis_index("c")
    s = lax.axis_index("s")
    w = c * 16 + s

    def zfill(i, _):
        zf[pl.ds(i * 16, 16)] = jnp.zeros((16,), jnp.float32)
        return 0

    lax.fori_loop(0, 200, zfill, 0)
    for j in range(4):
        pltpu.sync_copy(zf, accs.at[pl.ds(s * 12800 + j * 3200, 3200)])
    pltpu.sync_copy(srcg_hbm.at[pl.ds(w * GW, GW)], sb)
    pltpu.sync_copy(dst_hbm.at[pl.ds(w * GW, GW)], db)
    pltpu.sync_copy(a0_hbm.at[pl.ds(w * GW, GW)], a0b)
    pltpu.sync_copy(a1_hbm.at[pl.ds(w * GW, GW)], a1b)
    pltpu.sync_copy(dis_hbm, dis_t)
    plsc.subcore_barrier()

    def it(g, _):
        for k in range(8):
            sl = pl.ds(k * 16, 16)
            s16 = sb[g, sl]
            d16 = db[g, sl]
            c16 = a0b[g, sl] * 3 + a1b[g, sl]
            pay[sl] = plsc.load_gather(dis_t, [s16])
            a0b[g, sl] = d16 * SST + c16
        pltpu.sync_copy(pay, accs.at[a0b.at[g]], add=True)
        return 0

    lax.fori_loop(0, GW, it, 0)
    plsc.subcore_barrier()
    pltpu.sync_copy(accs.at[pl.ds(s * 12800, 12800)],
                    spart_hbm.at[pl.ds(c * SFLAT + s * 12800, 12800)])


@functools.partial(
    pl.kernel,
    out_type=(
        jax.ShapeDtypeStruct((ACCN, H), jnp.float32),
        jax.ShapeDtypeStruct((ACCN, H), jnp.float32),
    ),
    mesh=_mesh,
    compiler_params=_sc_params,
    scratch_types=[
        pltpu.VMEM((NBUF, 2, CH), jnp.int32),
        pltpu.VMEM((NBUF, CH, H), jnp.float32),
        pltpu.VMEM_SHARED((ACCN, H), jnp.float32),
        pltpu.SemaphoreType.DMA((NBUF,)),
        pltpu.SemaphoreType.DMA((NBUF,)),
        pltpu.SemaphoreType.DMA((NBUF,)),
    ],
)
def _sc_rowprop(uL_hbm, uR_hbm, idx_hbm, voutL_hbm, voutR_hbm,
                ibuf, rows, accv, gsem, ssem, isem):
    c = lax.axis_index("c")
    s = lax.axis_index("s")

    def zrow(i, _):
        for k in range(H // 16):
            rows[0, i, pl.ds(k * 16, 16)] = jnp.zeros((16,), jnp.float32)
        return 0

    lax.fori_loop(0, CH, zrow, 0)
    pltpu.sync_copy(rows.at[0], accv.at[pl.ds(s * 640, CH)])
    pltpu.sync_copy(rows.at[0], accv.at[pl.ds(s * 640 + CH, CH)])
    pltpu.sync_copy(rows.at[0, pl.ds(0, 128)],
                    accv.at[pl.ds(s * 640 + 2 * CH, 128)])
    plsc.subcore_barrier()

    def run(u_hbm):
        for b in range(NBUF):
            pltpu.sync_copy(idx_hbm.at[s * NCH + b], ibuf.at[b])
            pltpu.make_async_copy(u_hbm.at[ibuf.at[b, 0]], rows.at[b],
                                  gsem.at[b]).start()

        def round_body(gi, _):
            for b in range(NBUF):
                pltpu.make_async_copy(u_hbm.at[ibuf.at[b, 0]], rows.at[b],
                                      gsem.at[b]).wait()
                pltpu.make_async_copy(rows.at[b], accv.at[ibuf.at[b, 1]],
                                      ssem.at[b]).start(add=True)
            for b in range(NBUF):
                gn = s * NCH + gi * NBUF + b + NBUF
                pltpu.make_async_copy(rows.at[b], accv.at[ibuf.at[b, 1]],
                                      ssem.at[b]).wait()
                pltpu.make_async_copy(idx_hbm.at[gn], ibuf.at[b],
                                      isem.at[b]).start()
            for b in range(NBUF):
                gn = s * NCH + (gi + 1) * NBUF + b
                pltpu.make_async_copy(idx_hbm.at[gn], ibuf.at[b],
                                      isem.at[b]).wait()
                pltpu.make_async_copy(u_hbm.at[ibuf.at[b, 0]], rows.at[b],
                                      gsem.at[b]).start()
            return 0

        lax.fori_loop(0, NCH // NBUF - 1, round_body, 0)
        for b in range(NBUF):
            pltpu.make_async_copy(u_hbm.at[ibuf.at[b, 0]], rows.at[b],
                                  gsem.at[b]).wait()
            pltpu.make_async_copy(rows.at[b], accv.at[ibuf.at[b, 1]],
                                  ssem.at[b]).start(add=True)
        for b in range(NBUF):
            pltpu.make_async_copy(rows.at[b], accv.at[ibuf.at[b, 1]],
                                  ssem.at[b]).wait()

    @pl.when(c == 0)
    def _():
        run(uL_hbm)

    @pl.when(c == 1)
    def _():
        run(uR_hbm)

    plsc.subcore_barrier()
    @pl.when(c == 0)
    def _():
        pltpu.sync_copy(accv.at[pl.ds(s * 640, 640)],
                        voutL_hbm.at[pl.ds(s * 640, 640)])

    @pl.when(c == 1)
    def _():
        pltpu.sync_copy(accv.at[pl.ds(s * 640, 640)],
                        voutR_hbm.at[pl.ds(s * 640, 640)])


def _tc_dis_body(degp_ref, dis_ref):
    deg = degp_ref[0] + degp_ref[1]
    dis_ref[...] = jnp.where(deg > 0, lax.rsqrt(deg), 0.0)


_tc_dis = pl.pallas_call(
    _tc_dis_body,
    out_shape=jax.ShapeDtypeStruct((80, 128), jnp.float32),
    in_specs=[pl.BlockSpec((2, 80, 128), lambda: (0, 0, 0))],
    out_specs=pl.BlockSpec((80, 128), lambda: (0, 0)),
)


def _tc_u1_body(dis_ref, x_ref, u1l_ref, u1r_ref):
    u1 = dis_ref[...] * x_ref[...]
    u1l_ref[...] = u1[:, :H]
    u1r_ref[...] = u1[:, H:]


_tc_u1 = pl.pallas_call(
    _tc_u1_body,
    grid=(NBLK,),
    out_shape=(
        jax.ShapeDtypeStruct((N, H), jnp.float32),
        jax.ShapeDtypeStruct((N, H), jnp.float32),
    ),
    in_specs=[
        pl.BlockSpec((RB, 1), lambda b: (b, 0)),
        pl.BlockSpec((RB, D), lambda b: (b, 0)),
    ],
    out_specs=(
        pl.BlockSpec((RB, H), lambda b: (b, 0)),
        pl.BlockSpec((RB, H), lambda b: (b, 0)),
    ),
)


def _tc_mid_body(v1l_ref, v1r_ref, s0_ref, s1_ref, dis_ref, t_ref,
                 tx1_ref, u2l_ref, u2r_ref, b_ref):
    dis = dis_ref[...]
    bb = jnp.dot(s0_ref[...] + s1_ref[...], t_ref[...],
                 preferred_element_type=jnp.float32)
    b = dis * bb
    v1 = jnp.concatenate([v1l_ref[...], v1r_ref[...]], axis=1)
    tx1 = -(dis * v1 + b)
    u2 = dis * tx1
    tx1_ref[...] = tx1
    u2l_ref[...] = u2[:, :H]
    u2r_ref[...] = u2[:, H:]
    b_ref[...] = b


_tc_mid = pl.pallas_call(
    _tc_mid_body,
    grid=(NBLK,),
    out_shape=(
        jax.ShapeDtypeStruct((N, D), jnp.float32),
        jax.ShapeDtypeStruct((N, H), jnp.float32),
        jax.ShapeDtypeStruct((N, H), jnp.float32),
        jax.ShapeDtypeStruct((N, D), jnp.float32),
    ),
    in_specs=[
        pl.BlockSpec((RB, H), lambda b: (b, 0)),
        pl.BlockSpec((RB, H), lambda b: (b, 0)),
        pl.BlockSpec((RB, SST), lambda b: (b, 0)),
        pl.BlockSpec((RB, SST), lambda b: (b, 0)),
        pl.BlockSpec((RB, 1), lambda b: (b, 0)),
        pl.BlockSpec((SST, D), lambda b: (0, 0)),
    ],
    out_specs=(
        pl.BlockSpec((RB, D), lambda b: (b, 0)),
        pl.BlockSpec((RB, H), lambda b: (b, 0)),
        pl.BlockSpec((RB, H), lambda b: (b, 0)),
        pl.BlockSpec((RB, D), lambda b: (b, 0)),
    ),
)


def _tc_final_body(x_ref, tx1_ref, v2l_ref, v2r_ref, b_ref, dis_ref, w_ref,
                   bias_ref, out_ref):
    x = x_ref[...]
    tx1 = tx1_ref[...]
    v2 = jnp.concatenate([v2l_ref[...], v2r_ref[...]], axis=1)
    tx2 = -2.0 * (dis_ref[...] * v2 + b_ref[...]) - x
    acc = jnp.dot(x, w_ref[0], preferred_element_type=jnp.float32)
    acc += jnp.dot(tx1, w_ref[1], preferred_element_type=jnp.float32)
    acc += jnp.dot(tx2, w_ref[2], preferred_element_type=jnp.float32)
    out_ref[...] = acc + bias_ref[...]


_tc_final = pl.pallas_call(
    _tc_final_body,
    grid=(NBLK,),
    out_shape=jax.ShapeDtypeStruct((N, D), jnp.float32),
    in_specs=[
        pl.BlockSpec((RB, D), lambda b: (b, 0)),
        pl.BlockSpec((RB, D), lambda b: (b, 0)),
        pl.BlockSpec((RB, H), lambda b: (b, 0)),
        pl.BlockSpec((RB, H), lambda b: (b, 0)),
        pl.BlockSpec((RB, D), lambda b: (b, 0)),
        pl.BlockSpec((RB, 1), lambda b: (b, 0)),
        pl.BlockSpec((3, D, D), lambda b: (0, 0, 0)),
        pl.BlockSpec((1, D), lambda b: (0, 0)),
    ],
    out_specs=pl.BlockSpec((RB, D), lambda b: (b, 0)),
)


def kernel(x, edge_index, edge_attr, weight, bias, emb1, emb2):
    src = edge_index[0]
    dst = edge_index[1]
    a0 = edge_attr[:, 0]
    a1 = edge_attr[:, 1]
    P = EP - E
    zpad = jnp.zeros((P,), jnp.int32)
    gpad = jnp.full((P,), GB, jnp.int32)
    srcg = jnp.concatenate([src, zpad]).reshape(EP // B, B)
    srcd = jnp.concatenate([src, gpad]).reshape(EP // B, B)
    dstp = jnp.concatenate([dst, gpad]).reshape(EP // B, B)
    a0p = jnp.concatenate([a0, zpad]).reshape(EP // B, B)
    a1p = jnp.concatenate([a1, zpad]).reshape(EP // B, B)
    t_full = (emb1[:, None, :] + emb2[None, :, :]).reshape(18, D)
    tpad = jnp.concatenate([t_full, jnp.zeros((SST - 18, D), jnp.float32)],
                           axis=0)

    degp = _sc_deg(srcd)
    disf = _tc_dis(degp.reshape(2, 80, 128))
    dis_flat = disf.reshape(ACCN)
    discol = dis_flat[:N][:, None]
    u1L, u1R = _tc_u1(discol, x)
    spart = _sc_sprime(dis_flat, srcg, dstp, a0p, a1p)
    s2 = spart.reshape(2, ACCN, SST)
    packed = jnp.stack([srcg.reshape(-1, CH), dstp.reshape(-1, CH)], axis=1)
    v1L, v1R = _sc_rowprop(u1L, u1R, packed)
    tx1, u2L, u2R, b = _tc_mid(v1L[:N], v1R[:N], s2[0, :N], s2[1, :N],
                               discol, tpad)
    v2L, v2R = _sc_rowprop(u2L, u2R, packed)
    out = _tc_final(x, tx1, v2L[:N], v2R[:N], b, discol, weight,
                    bias[None, :])
    return out
